# baseline (speedup 1.0000x reference)
import functools
import math

import jax
import jax.numpy as jnp
from jax import lax
from jax.experimental import pallas as pl
from jax.experimental.pallas import tpu as pltpu
from jax.experimental.pallas import tpu_sc as plsc

EMB = 64
SCALE = math.sqrt(EMB)

NC = 2
NS = 16
NW = NC * NS
LANES = 16

CH = 128
NBUF = 4
DB = 80
NDB = 4


def _make_depad(V):
    assert V % DB == 0 and DB % 8 == 0
    nblk = V // DB
    nslot = -(-(-(-nblk // NW) // NDB)) * NDB
    nslot = ((nblk + NW - 1) // NW + NDB - 1) // NDB * NDB
    mesh = plsc.VectorSubcoreMesh(
        core_axis_name="c", subcore_axis_name="s", num_cores=NC, num_subcores=NS
    )

    @functools.partial(
        pl.kernel,
        out_type=jax.ShapeDtypeStruct((V // 2, 2 * EMB), jnp.float32),
        mesh=mesh,
        compiler_params=pltpu.CompilerParams(
            use_tc_tiling_on_sc=True, needs_layout_passes=False),
        scratch_types=[
            pltpu.VMEM((NDB, DB, EMB), jnp.float32),
            pltpu.VMEM((NDB, DB // 2, 2 * EMB), jnp.float32),
            pltpu.SemaphoreType.DMA((NDB,)),
            pltpu.SemaphoreType.DMA((NDB,)),
        ],
    )
    def depad_kernel(table_hbm, out_hbm, bufin, bufout, isem, osem):
        wid = lax.axis_index("s") * NC + lax.axis_index("c")

        def blk(g):
            return wid + g * NW

        def in_copy(g, p):
            i = blk(g)
            return pltpu.make_async_copy(
                table_hbm.at[pl.ds(i * DB, DB)], bufin.at[p], isem.at[p])

        def out_copy(g, p):
            i = blk(g)
            return pltpu.make_async_copy(
                bufout.at[p],
                out_hbm.at[pl.ds(i * (DB // 2), DB // 2)], osem.at[p])

        for p in range(NDB):

            @pl.when(blk(p) < nblk)
            def _():
                in_copy(p, p).start()

        @pl.loop(0, nslot, step=NDB)
        def _grp(g0):
            for p in range(NDB):
                g = g0 + p

                @pl.when((g >= NDB) & (blk(g - NDB) < nblk))
                def _():
                    out_copy(g - NDB, p).wait()

                @pl.when(blk(g) < nblk)
                def _():
                    in_copy(g, p).wait()
                    src = bufin.at[p]
                    dst = bufout.at[p]

                    @plsc.parallel_loop(0, DB // 2, unroll=2)
                    def _row(r):
                        for h in range(2):
                            for c in range(EMB // LANES):
                                dst[r, pl.ds(h * EMB + c * LANES, LANES)] = (
                                    src[2 * r + h, pl.ds(c * LANES, LANES)])

                    out_copy(g, p).start()

                    @pl.when(blk(g + NDB) < nblk)
                    def _():
                        in_copy(g + NDB, p).start()

        for p in range(NDB):
            g = nslot - NDB + p

            @pl.when(blk(g) < nblk)
            def _():
                out_copy(g, p).wait()

    return depad_kernel


def _make_gather(B):
    assert B % (NW * CH) == 0
    b_per_w = B // NW
    nchunk = b_per_w // CH
    mesh = plsc.VectorSubcoreMesh(
        core_axis_name="c", subcore_axis_name="s", num_cores=NC, num_subcores=NS
    )

    @functools.partial(
        pl.kernel,
        out_type=jax.ShapeDtypeStruct((B, 2 * EMB), jnp.float32),
        mesh=mesh,
        compiler_params=pltpu.CompilerParams(
            use_tc_tiling_on_sc=False, needs_layout_passes=False),
        scratch_types=[
            pltpu.VMEM((nchunk, CH), jnp.int32),
            pltpu.VMEM((NBUF, CH, EMB), jnp.float32),
            pltpu.VMEM((NBUF, CH, EMB), jnp.float32),
            pltpu.SemaphoreType.DMA((NBUF,)),
            pltpu.SemaphoreType.DMA((NBUF,)),
        ],
    )
    def gather_kernel(tok_hbm, table_hbm, out_hbm, idx_v, gbuf, sbuf,
                      gsem, ssem):
        wid = lax.axis_index("s") * NC + lax.axis_index("c")
        base = wid * b_per_w
        pltpu.sync_copy(tok_hbm.at[wid], idx_v)

        def gather_copy(j, b):
            return pltpu.make_async_copy(
                table_hbm.at[idx_v.at[j]], gbuf.at[b], gsem.at[b])

        def store_copy(j, b):
            return pltpu.make_async_copy(
                sbuf.at[b],
                out_hbm.at[pl.ds(base + j * CH, CH), pl.ds(0, EMB)],
                ssem.at[b])

        for b in range(NBUF):
            gather_copy(b, b).start()

        @pl.loop(0, nchunk, step=NBUF)
        def _group(g):
            for b in range(NBUF):
                j = g + b
                gather_copy(j, b).wait()

                @pl.when(j >= NBUF)
                def _drain():
                    store_copy(j - NBUF, b).wait()

                src = gbuf.at[b]
                dst = sbuf.at[b]

                @plsc.parallel_loop(0, CH, unroll=4)
                def _scale(r):
                    for c in range(EMB // LANES):
                        sl = pl.ds(c * LANES, LANES)
                        dst[r, sl] = src[r, sl] * SCALE

                nj = j + NBUF

                @pl.when(nj < nchunk)
                def _prefetch():
                    gather_copy(nj, b).start()

                store_copy(j, b).start()

        for b in range(NBUF):
            store_copy(nchunk - NBUF + b, b).wait()

    return gather_kernel


def kernel(tokens, table):
    R, T = tokens.shape
    B = R * T
    V = table.shape[0]
    tok = tokens.astype(jnp.int32).reshape(NW, B // NW // CH, CH)
    tab_compact = _make_depad(V)(table).reshape(V, EMB)
    padded = _make_gather(B)(tok, tab_compact)
    return padded.reshape(R, T, 2 * EMB)[:, :, :EMB]
